# baseline (speedup 1.0000x reference)
import functools

import jax
import jax.numpy as jnp
from jax import lax
from jax.experimental import pallas as pl
from jax.experimental.pallas import tpu as pltpu
from jax.experimental.pallas import tpu_sc as plsc

VOCAB = 100000
EMBED = 128
BATCH = 16
SEQ = 2048
GATES = 4 * EMBED

SC_CORES = 2
SC_SUBCORES = 16
NW = SC_CORES * SC_SUBCORES
TOTAL = SEQ * BATCH
GCHUNK = 256

NSEG = 2
SEGT = SEQ // NSEG
SEGROWS = TOTAL // NSEG
UNROLL = 32


def _sc_gather(table, flat_ids, start):
    mesh = plsc.VectorSubcoreMesh(core_axis_name="c", subcore_axis_name="s")
    rows_per_w = SEGROWS // NW

    @functools.partial(
        pl.kernel,
        out_type=jax.ShapeDtypeStruct((SEGROWS, EMBED), jnp.float32),
        mesh=mesh,
        scratch_types=[
            pltpu.VMEM((GCHUNK,), jnp.int32),
            pltpu.VMEM((GCHUNK, EMBED), jnp.float32),
            pltpu.SemaphoreType.DMA,
        ],
    )
    def gather_kernel(table_hbm, idx_hbm, out_hbm, idx_v, rows_v, sem):
        wid = lax.axis_index("s") * SC_CORES + lax.axis_index("c")
        base = wid * rows_per_w

        @pl.loop(0, rows_per_w, step=GCHUNK)
        def _(off):
            pltpu.sync_copy(idx_hbm.at[pl.ds(start + base + off, GCHUNK)],
                            idx_v)
            pltpu.async_copy(table_hbm.at[idx_v], rows_v, sem).wait()
            pltpu.sync_copy(rows_v, out_hbm.at[pl.ds(base + off, GCHUNK)])

    return gather_kernel(table, flat_ids)


def _make_lstm_seg(t_start, last):
    t_end = t_start + SEGT

    def body(emb_ref, wcat_ref, bias_ref, len_ref, h_in, c_in, acc_in,
             h_out, c_out, acc_out):
        maxlen = jnp.max(len_ref[...])
        bound = jnp.clip(maxlen, t_start, t_end) - t_start
        nblocks = (bound + UNROLL - 1) // UNROLL

        def step(t, hp, c, acc):
            xt = emb_ref[t - t_start].astype(jnp.bfloat16)
            hx = jnp.concatenate([xt, hp.astype(jnp.bfloat16)], axis=1)
            g = jnp.dot(hx, wcat_ref[...],
                        preferred_element_type=jnp.float32) + bias_ref[...]
            t_i = jnp.tanh(g[:, 0:EMBED])
            t_f = jnp.tanh(g[:, EMBED:2 * EMBED])
            g_g = jnp.tanh(g[:, 2 * EMBED:3 * EMBED])
            t_o = jnp.tanh(g[:, 3 * EMBED:4 * EMBED])
            a1 = t_f * c + c
            a2 = t_i * g_g + g_g
            c = 0.5 * a1 + 0.5 * a2
            tau = jnp.tanh(c)
            hp = t_o * tau + tau
            acc = jnp.where(len_ref[...] == t + 1, hp, acc)
            return hp, c, acc

        def block(bi, carry):
            hp, c, acc = carry
            t0 = t_start + bi * UNROLL
            for k in range(UNROLL):
                hp, c, acc = step(t0 + k, hp, c, acc)
            return hp, c, acc

        hp, c, acc = lax.fori_loop(
            0, nblocks, block, (h_in[...], c_in[...], acc_in[...]))
        h_out[...] = hp
        c_out[...] = c
        acc_out[...] = 0.5 * acc if last else acc

    small = pl.BlockSpec((BATCH, EMBED), lambda: (0, 0))
    st = jax.ShapeDtypeStruct((BATCH, EMBED), jnp.float32)
    return pl.pallas_call(
        body,
        in_specs=[
            pl.BlockSpec((SEGT, BATCH, EMBED), lambda: (0, 0, 0)),
            pl.BlockSpec((2 * EMBED, GATES), lambda: (0, 0)),
            pl.BlockSpec((1, GATES), lambda: (0, 0)),
            small, small, small, small,
        ],
        out_specs=(small, small, small),
        out_shape=(st, st, st),
    )


def kernel(token_ids, seq_lengths, table, W_ih, W_hh, b_ih, b_hh):
    flat_ids = token_ids.astype(jnp.int32).T.reshape(TOTAL)
    halve = jnp.concatenate(
        [jnp.full((EMBED,), 0.5), jnp.full((EMBED,), 0.5),
         jnp.ones((EMBED,)), jnp.full((EMBED,), 0.5)]).astype(jnp.float32)
    rowscale = jnp.concatenate(
        [jnp.ones((EMBED,)), jnp.full((EMBED,), 0.5)]).astype(jnp.float32)
    wcat = (jnp.concatenate([W_ih.T, W_hh.T], axis=0)
            * halve[None, :] * rowscale[:, None]).astype(jnp.bfloat16)
    bias = ((b_ih + b_hh) * halve).reshape(1, GATES)
    lens = jnp.broadcast_to(
        seq_lengths.astype(jnp.int32)[:, None], (BATCH, EMBED))

    z = jnp.zeros((BATCH, EMBED), jnp.float32)
    h, c, acc = z, z, z
    for seg in range(NSEG):
        emb = _sc_gather(table, flat_ids, seg * SEGROWS)
        emb = emb.reshape(SEGT, BATCH, EMBED)
        h, c, acc = _make_lstm_seg(seg * SEGT, seg == NSEG - 1)(
            emb, wcat, bias, lens, h, c, acc)
    return acc


# speedup vs baseline: 1.0093x; 1.0093x over previous
import functools

import jax
import jax.numpy as jnp
from jax import lax
from jax.experimental import pallas as pl
from jax.experimental.pallas import tpu as pltpu
from jax.experimental.pallas import tpu_sc as plsc

VOCAB = 100000
EMBED = 128
BATCH = 16
SEQ = 2048
GATES = 4 * EMBED

SC_CORES = 2
SC_SUBCORES = 16
NW = SC_CORES * SC_SUBCORES
TOTAL = SEQ * BATCH
GCHUNK = 256

NSEG = 4
SEGT = SEQ // NSEG
SEGROWS = TOTAL // NSEG
UNROLL = 16


def _sc_gather(table, flat_ids, start):
    mesh = plsc.VectorSubcoreMesh(core_axis_name="c", subcore_axis_name="s")
    rows_per_w = SEGROWS // NW

    @functools.partial(
        pl.kernel,
        out_type=jax.ShapeDtypeStruct((SEGROWS, EMBED), jnp.float32),
        mesh=mesh,
        scratch_types=[
            pltpu.VMEM((GCHUNK,), jnp.int32),
            pltpu.VMEM((GCHUNK, EMBED), jnp.float32),
            pltpu.SemaphoreType.DMA,
        ],
    )
    def gather_kernel(table_hbm, idx_hbm, out_hbm, idx_v, rows_v, sem):
        wid = lax.axis_index("s") * SC_CORES + lax.axis_index("c")
        base = wid * rows_per_w

        @pl.loop(0, rows_per_w, step=GCHUNK)
        def _(off):
            pltpu.sync_copy(idx_hbm.at[pl.ds(start + base + off, GCHUNK)],
                            idx_v)
            pltpu.async_copy(table_hbm.at[idx_v], rows_v, sem).wait()
            pltpu.sync_copy(rows_v, out_hbm.at[pl.ds(base + off, GCHUNK)])

    return gather_kernel(table, flat_ids)


def _make_lstm_seg(t_start, last):
    t_end = t_start + SEGT

    def body(emb_ref, wcat_ref, bias_ref, len_ref, h_in, c_in, acc_in,
             h_out, c_out, acc_out):
        maxlen = jnp.max(len_ref[...])
        bound = jnp.clip(maxlen, t_start, t_end) - t_start
        nblocks = (bound + UNROLL - 1) // UNROLL

        def step(t, hp, c, acc):
            xt = emb_ref[t - t_start].astype(jnp.bfloat16)
            hx = jnp.concatenate([xt, hp.astype(jnp.bfloat16)], axis=1)
            g = jnp.dot(hx, wcat_ref[...],
                        preferred_element_type=jnp.float32) + bias_ref[...]
            i_g = 0.5 * jnp.tanh(g[:, 0:EMBED]) + 0.5
            f_g = 0.5 * jnp.tanh(g[:, EMBED:2 * EMBED]) + 0.5
            g_g = jnp.tanh(g[:, 2 * EMBED:3 * EMBED])
            o_g = 0.5 * jnp.tanh(g[:, 3 * EMBED:4 * EMBED]) + 0.5
            c = f_g * c + i_g * g_g
            hp = o_g * jnp.tanh(c)
            acc = jnp.where(len_ref[...] == t + 1, hp, acc)
            return hp, c, acc

        def block(bi, carry):
            hp, c, acc = carry
            t0 = t_start + bi * UNROLL
            for k in range(UNROLL):
                hp, c, acc = step(t0 + k, hp, c, acc)
            return hp, c, acc

        hp, c, acc = lax.fori_loop(
            0, nblocks, block, (h_in[...], c_in[...], acc_in[...]))
        h_out[...] = hp
        c_out[...] = c
        acc_out[...] = acc

    small = pl.BlockSpec((BATCH, EMBED), lambda: (0, 0))
    st = jax.ShapeDtypeStruct((BATCH, EMBED), jnp.float32)
    return pl.pallas_call(
        body,
        in_specs=[
            pl.BlockSpec((SEGT, BATCH, EMBED), lambda: (0, 0, 0)),
            pl.BlockSpec((2 * EMBED, GATES), lambda: (0, 0)),
            pl.BlockSpec((1, GATES), lambda: (0, 0)),
            small, small, small, small,
        ],
        out_specs=(small, small, small),
        out_shape=(st, st, st),
    )


def kernel(token_ids, seq_lengths, table, W_ih, W_hh, b_ih, b_hh):
    flat_ids = token_ids.astype(jnp.int32).T.reshape(TOTAL)
    halve = jnp.concatenate(
        [jnp.full((EMBED,), 0.5), jnp.full((EMBED,), 0.5),
         jnp.ones((EMBED,)), jnp.full((EMBED,), 0.5)]).astype(jnp.float32)
    wcat = (jnp.concatenate([W_ih.T, W_hh.T], axis=0)
            * halve[None, :]).astype(jnp.bfloat16)
    bias = ((b_ih + b_hh) * halve).reshape(1, GATES)
    lens = jnp.broadcast_to(
        seq_lengths.astype(jnp.int32)[:, None], (BATCH, EMBED))

    z = jnp.zeros((BATCH, EMBED), jnp.float32)
    h, c, acc = z, z, z
    for seg in range(NSEG):
        emb = _sc_gather(table, flat_ids, seg * SEGROWS)
        emb = emb.reshape(SEGT, BATCH, EMBED)
        h, c, acc = _make_lstm_seg(seg * SEGT, seg == NSEG - 1)(
            emb, wcat, bias, lens, h, c, acc)
    return acc
